# Initial kernel scaffold; baseline (speedup 1.0000x reference)
import jax
import jax.numpy as jnp
from jax.experimental import pallas as pl


def kernel(data, segment_ids, num_segments):
    raise NotImplementedError("write your pallas kernel here")


# speedup vs baseline: 5.4309x; 5.4309x over previous
import functools

import jax
import jax.numpy as jnp
from jax import lax
from jax.experimental import pallas as pl
from jax.experimental.pallas import tpu as pltpu
from jax.experimental.pallas import tpu_sc as plsc

N_TOTAL = 6400000
NUM_SEG = 100000
LANES = 128
BLK = 12800
NBLK = N_TOTAL // BLK
NWORK = 32
KMAX = (NBLK + NWORK - 1) // NWORK
SPAD = 100352
SEG_SLICE = SPAD // 16


def _sc_segment_partials(data, ids):
    mesh = plsc.VectorSubcoreMesh(core_axis_name="c", subcore_axis_name="s")

    @functools.partial(
        pl.kernel,
        out_type=jax.ShapeDtypeStruct((2, SPAD), jnp.float32),
        mesh=mesh,
        scratch_types=[
            pltpu.VMEM((BLK,), jnp.float32),
            pltpu.VMEM((BLK,), jnp.int32),
            pltpu.VMEM((SEG_SLICE,), jnp.float32),
            pltpu.VMEM_SHARED((SPAD,), jnp.float32),
        ],
    )
    def k(data_hbm, ids_hbm, out_hbm, dbuf, ibuf, zbuf, acc):
        c = lax.axis_index("c")
        s = lax.axis_index("s")
        w = c * 16 + s

        def zinit(i, carry):
            zbuf[pl.ds(i * 16, 16)] = jnp.zeros((16,), jnp.float32)
            return carry

        lax.fori_loop(0, SEG_SLICE // 16, zinit, 0)
        pltpu.sync_copy(zbuf, acc.at[pl.ds(s * SEG_SLICE, SEG_SLICE)])
        plsc.subcore_barrier()

        def body(kk, carry):
            b = w + NWORK * kk

            @pl.when(b < NBLK)
            def _():
                e0 = b * BLK
                pltpu.sync_copy(data_hbm.at[pl.ds(e0, BLK)], dbuf)
                pltpu.sync_copy(ids_hbm.at[pl.ds(e0, BLK)], ibuf)
                pltpu.sync_copy(dbuf, acc.at[ibuf], add=True)

            return carry

        lax.fori_loop(0, KMAX, body, 0)
        plsc.subcore_barrier()

        pltpu.sync_copy(
            acc.at[pl.ds(s * SEG_SLICE, SEG_SLICE)],
            out_hbm.at[c, pl.ds(s * SEG_SLICE, SEG_SLICE)],
        )

    return k(data, ids)


def _tc_combine(partials):
    x = partials.reshape(2, SPAD // LANES, LANES)

    def body(x_ref, o_ref):
        o_ref[...] = x_ref[0] + x_ref[1]

    out = pl.pallas_call(
        body,
        out_shape=jax.ShapeDtypeStruct((SPAD // LANES, LANES), jnp.float32),
    )(x)
    return out.reshape(SPAD)


def kernel(data, segment_ids, num_segments):
    partials = _sc_segment_partials(data, segment_ids)
    return _tc_combine(partials)[:NUM_SEG]
